# Initial kernel scaffold; baseline (speedup 1.0000x reference)
import jax
import jax.numpy as jnp
from jax.experimental import pallas as pl


def kernel(h, edge_index, W, a, nw):
    raise NotImplementedError("write your pallas kernel here")


# speedup vs baseline: 16.9706x; 16.9706x over previous
import functools

import jax
import jax.numpy as jnp
from jax import lax
from jax.experimental import pallas as pl
from jax.experimental.pallas import tpu as pltpu
from jax.experimental.pallas import tpu_sc as plsc

NC = 2
NS = 16
L = 16
NWK = NC * NS
ALPHA = 0.2
CK = 80
ER = 2500
EC = 128
EBLK = 2500
PR = 8


def _mesh():
    return plsc.VectorSubcoreMesh(
        core_axis_name="c", subcore_axis_name="s", num_cores=NC, num_subcores=NS
    )


def _tc_pre(h, W, a1, a2):
    n, d_in = h.shape
    d = W.shape[1]
    rb = 1000

    def body(h_ref, w_ref, a1_ref, a2_ref, wh_ref, s1_ref, s2_ref):
        wh = jnp.dot(h_ref[...], w_ref[...], preferred_element_type=jnp.float32)
        wh_ref[...] = wh
        s1_ref[...] = jnp.dot(wh, a1_ref[...], preferred_element_type=jnp.float32)
        s2_ref[...] = jnp.dot(wh, a2_ref[...], preferred_element_type=jnp.float32)

    return pl.pallas_call(
        body,
        grid=(n // rb,),
        in_specs=[
            pl.BlockSpec((rb, d_in), lambda i: (i, 0)),
            pl.BlockSpec((d_in, d), lambda i: (0, 0)),
            pl.BlockSpec((d, 1), lambda i: (0, 0)),
            pl.BlockSpec((d, 1), lambda i: (0, 0)),
        ],
        out_specs=[
            pl.BlockSpec((rb, d), lambda i: (i, 0)),
            pl.BlockSpec((rb, 1), lambda i: (i, 0)),
            pl.BlockSpec((rb, 1), lambda i: (i, 0)),
        ],
        out_shape=[
            jax.ShapeDtypeStruct((n, d), jnp.float32),
            jax.ShapeDtypeStruct((n, 1), jnp.float32),
            jax.ShapeDtypeStruct((n, 1), jnp.float32),
        ],
    )(h, W, a1, a2)


def _sc_edge(s1, s2, src, dst):
    e = src.shape[0]
    epw = e // NWK
    n = s1.shape[0]

    @functools.partial(
        pl.kernel,
        mesh=_mesh(),
        compiler_params=pltpu.CompilerParams(needs_layout_passes=False),
        out_type=jax.ShapeDtypeStruct((e,), jnp.float32),
        scratch_types=[
            pltpu.VMEM((n,), jnp.float32),
            pltpu.VMEM((n,), jnp.float32),
            pltpu.VMEM((epw,), jnp.int32),
            pltpu.VMEM((epw,), jnp.int32),
            pltpu.VMEM((epw,), jnp.float32),
        ],
    )
    def k(s1_hbm, s2_hbm, src_hbm, dst_hbm, lk_hbm,
          s1_v, s2_v, src_v, dst_v, lk_v):
        wid = lax.axis_index("s") * NC + lax.axis_index("c")
        base = wid * epw
        pltpu.sync_copy(s1_hbm, s1_v)
        pltpu.sync_copy(s2_hbm, s2_v)
        pltpu.sync_copy(src_hbm.at[pl.ds(base, epw)], src_v)
        pltpu.sync_copy(dst_hbm.at[pl.ds(base, epw)], dst_v)

        def step(i, carry):
            x1 = plsc.load_gather(s1_v, [src_v[pl.ds(i * L, L)]])
            x2 = plsc.load_gather(s2_v, [dst_v[pl.ds(i * L, L)]])
            ev = x1 + x2
            lk_v[pl.ds(i * L, L)] = jnp.where(ev > 0, -ev, -ALPHA * ev)
            return carry

        lax.fori_loop(0, epw // L, step, 0)
        pltpu.sync_copy(lk_v, lk_hbm.at[pl.ds(base, epw)])

    return k(s1, s2, src, dst)


def _tc_softmax_stats(lk2):
    nblk = ER // EBLK

    def body(lk_ref, m_ref, s_ref):
        x = jnp.exp(lk_ref[...])
        m = jnp.max(x, axis=0, keepdims=True)
        s = jnp.sum(jnp.exp(x - m), axis=0, keepdims=True)
        m_ref[...] = jnp.broadcast_to(m, (PR, EC))
        s_ref[...] = jnp.broadcast_to(s, (PR, EC))

    return pl.pallas_call(
        body,
        grid=(nblk,),
        in_specs=[pl.BlockSpec((EBLK, EC), lambda i: (i, 0))],
        out_specs=[pl.BlockSpec((PR, EC), lambda i: (i, 0)),
                   pl.BlockSpec((PR, EC), lambda i: (i, 0))],
        out_shape=[jax.ShapeDtypeStruct((nblk * PR, EC), jnp.float32),
                   jax.ShapeDtypeStruct((nblk * PR, EC), jnp.float32)],
    )(lk2)


def _tc_vals(lk2, mp):
    nblk = ER // EBLK

    def body(lk_ref, m_ref, v_ref):
        gmax = jnp.max(m_ref[...])
        v_ref[...] = jnp.exp(jnp.exp(lk_ref[...]) - gmax)

    return pl.pallas_call(
        body,
        grid=(nblk,),
        in_specs=[pl.BlockSpec((EBLK, EC), lambda i: (i, 0)),
                  pl.BlockSpec(mp.shape, lambda i: (0, 0))],
        out_specs=pl.BlockSpec((EBLK, EC), lambda i: (i, 0)),
        out_shape=jax.ShapeDtypeStruct((ER, EC), jnp.float32),
    )(lk2, mp)


def _sc_spmm(wh, src3, dst, vals, zeros):
    n, d = wh.shape
    nch = src3.shape[1]
    epw = nch * CK
    rpt = (n // NS) // 8 * 8
    tail = n - NS * rpt

    @functools.partial(
        pl.kernel,
        mesh=_mesh(),
        compiler_params=pltpu.CompilerParams(needs_layout_passes=False),
        out_type=jax.ShapeDtypeStruct((NC, n, d), jnp.float32),
        scratch_types=[
            pltpu.VMEM((nch, CK), jnp.int32),
            pltpu.VMEM((epw,), jnp.int32),
            pltpu.VMEM((epw,), jnp.float32),
            pltpu.VMEM((CK, d), jnp.float32),
            pltpu.VMEM_SHARED((n, d), jnp.float32),
            pltpu.SemaphoreType.DMA,
        ],
    )
    def k(wh_hbm, src_hbm, dst_hbm, vals_hbm, z_hbm, out_hbm,
          src_v, dst_v, vals_v, rows_v, acc, sem):
        cid = lax.axis_index("c")
        sid = lax.axis_index("s")
        wid = sid * NC + cid
        row0 = pl.multiple_of(sid * rpt, 8)
        pltpu.sync_copy(z_hbm.at[pl.ds(row0, rpt)], acc.at[pl.ds(row0, rpt)])

        @pl.when(sid == 0)
        def _zero_tail():
            pltpu.sync_copy(z_hbm.at[pl.ds(NS * rpt, tail)],
                            acc.at[pl.ds(NS * rpt, tail)])

        pltpu.sync_copy(src_hbm.at[wid], src_v)
        pltpu.sync_copy(dst_hbm.at[pl.ds(wid * epw, epw)], dst_v)
        pltpu.sync_copy(vals_hbm.at[pl.ds(wid * epw, epw)], vals_v)

        plsc.subcore_barrier()

        def chunk(c, _):
            pltpu.async_copy(wh_hbm.at[dst_v.at[pl.ds(c * CK, CK)]], rows_v,
                             sem).wait()

            def gstep(g, _):
                vv = vals_v[pl.ds(c * CK + g * L, L)]
                for i in range(L):
                    v = vv[i]
                    r = g * L + i
                    for q in range(d // L):
                        rows_v[r, pl.ds(q * L, L)] = rows_v[r, pl.ds(q * L, L)] * v
                return 0

            lax.fori_loop(0, CK // L, gstep, 0)
            pltpu.sync_copy(rows_v, acc.at[src_v.at[c]], add=True)
            return 0

        lax.fori_loop(0, nch, chunk, 0)
        plsc.subcore_barrier()
        pltpu.sync_copy(acc.at[pl.ds(row0, rpt)],
                        out_hbm.at[cid, pl.ds(row0, rpt)])

        @pl.when(sid == 0)
        def _dump_tail():
            pltpu.sync_copy(acc.at[pl.ds(NS * rpt, tail)],
                            out_hbm.at[cid, pl.ds(NS * rpt, tail)])

    return k(wh, src3, dst, vals, zeros)


def _tc_post(p0, p1, mp, sp):
    n, d = p0.shape
    rb = 1000

    def body(a_ref, b_ref, m_ref, s_ref, o_ref):
        gmax = jnp.max(m_ref[...])
        s_tot = jnp.sum(s_ref[...] * jnp.exp(m_ref[...] - gmax)) * (1.0 / PR)
        x = (a_ref[...] + b_ref[...]) / s_tot
        o_ref[...] = jnp.where(x > 0, x, jnp.exp(x) - 1.0)

    return pl.pallas_call(
        body,
        grid=(n // rb,),
        in_specs=[pl.BlockSpec((rb, d), lambda i: (i, 0)),
                  pl.BlockSpec((rb, d), lambda i: (i, 0)),
                  pl.BlockSpec(mp.shape, lambda i: (0, 0)),
                  pl.BlockSpec(sp.shape, lambda i: (0, 0))],
        out_specs=pl.BlockSpec((rb, d), lambda i: (i, 0)),
        out_shape=jax.ShapeDtypeStruct((n, d), jnp.float32),
    )(p0, p1, mp, sp)


def kernel(h, edge_index, W, a, nw):
    del nw
    n, _ = h.shape
    d = W.shape[1]
    e = edge_index.shape[1]
    a1 = a[0:d]
    a2 = a[d:2 * d]
    wh, s1, s2 = _tc_pre(h, W, a1, a2)
    s1 = s1.reshape(n)
    s2 = s2.reshape(n)
    src = edge_index[0]
    dst = edge_index[1]
    lk = _sc_edge(s1, s2, src, dst)
    lk2 = lk.reshape(ER, EC)
    mp, sp = _tc_softmax_stats(lk2)
    vals = _tc_vals(lk2, mp).reshape(e)
    nch = e // NWK // CK
    src3 = src.reshape(NWK, nch, CK)
    zeros = jnp.zeros((n, d), jnp.float32)
    acc2 = _sc_spmm(wh, src3, dst, vals, zeros)
    return _tc_post(acc2[0], acc2[1], mp, sp)


# speedup vs baseline: 23.6742x; 1.3950x over previous
import functools

import jax
import jax.numpy as jnp
from jax import lax
from jax.experimental import pallas as pl
from jax.experimental.pallas import tpu as pltpu
from jax.experimental.pallas import tpu_sc as plsc

NC = 2
NS = 16
L = 16
NWK = NC * NS
ALPHA = 0.2
CK = 80
ER = 2500
EC = 128
EBLK = 2500
PR = 8


def _mesh():
    return plsc.VectorSubcoreMesh(
        core_axis_name="c", subcore_axis_name="s", num_cores=NC, num_subcores=NS
    )


def _tc_pre(h, W, a1, a2):
    n, d_in = h.shape
    d = W.shape[1]
    rb = 1000

    def body(h_ref, w_ref, a1_ref, a2_ref, wh_ref, s1_ref, s2_ref):
        wh = jnp.dot(h_ref[...], w_ref[...], preferred_element_type=jnp.float32)
        wh_ref[...] = wh
        s1_ref[...] = jnp.dot(wh, a1_ref[...], preferred_element_type=jnp.float32)
        s2_ref[...] = jnp.dot(wh, a2_ref[...], preferred_element_type=jnp.float32)

    return pl.pallas_call(
        body,
        grid=(n // rb,),
        in_specs=[
            pl.BlockSpec((rb, d_in), lambda i: (i, 0)),
            pl.BlockSpec((d_in, d), lambda i: (0, 0)),
            pl.BlockSpec((d, 1), lambda i: (0, 0)),
            pl.BlockSpec((d, 1), lambda i: (0, 0)),
        ],
        out_specs=[
            pl.BlockSpec((rb, d), lambda i: (i, 0)),
            pl.BlockSpec((rb, 1), lambda i: (i, 0)),
            pl.BlockSpec((rb, 1), lambda i: (i, 0)),
        ],
        out_shape=[
            jax.ShapeDtypeStruct((n, d), jnp.float32),
            jax.ShapeDtypeStruct((n, 1), jnp.float32),
            jax.ShapeDtypeStruct((n, 1), jnp.float32),
        ],
    )(h, W, a1, a2)


def _sc_edge(s1, s2, src, dst):
    e = src.shape[0]
    epw = e // NWK
    n = s1.shape[0]

    @functools.partial(
        pl.kernel,
        mesh=_mesh(),
        compiler_params=pltpu.CompilerParams(needs_layout_passes=False),
        out_type=jax.ShapeDtypeStruct((e,), jnp.float32),
        scratch_types=[
            pltpu.VMEM((n,), jnp.float32),
            pltpu.VMEM((n,), jnp.float32),
            pltpu.VMEM((epw,), jnp.int32),
            pltpu.VMEM((epw,), jnp.int32),
            pltpu.VMEM((epw,), jnp.float32),
        ],
    )
    def k(s1_hbm, s2_hbm, src_hbm, dst_hbm, lk_hbm,
          s1_v, s2_v, src_v, dst_v, lk_v):
        wid = lax.axis_index("s") * NC + lax.axis_index("c")
        base = wid * epw
        pltpu.sync_copy(s1_hbm, s1_v)
        pltpu.sync_copy(s2_hbm, s2_v)
        pltpu.sync_copy(src_hbm.at[pl.ds(base, epw)], src_v)
        pltpu.sync_copy(dst_hbm.at[pl.ds(base, epw)], dst_v)

        def step(i, carry):
            x1 = plsc.load_gather(s1_v, [src_v[pl.ds(i * L, L)]])
            x2 = plsc.load_gather(s2_v, [dst_v[pl.ds(i * L, L)]])
            ev = x1 + x2
            lk_v[pl.ds(i * L, L)] = jnp.where(ev > 0, -ev, -ALPHA * ev)
            return carry

        lax.fori_loop(0, epw // L, step, 0)
        pltpu.sync_copy(lk_v, lk_hbm.at[pl.ds(base, epw)])

    return k(s1, s2, src, dst)


def _tc_vals(lk2):
    def body(lk_ref, v_ref, s_ref):
        x = jnp.exp(lk_ref[...])
        gmax = jnp.max(x)
        v = jnp.exp(x - gmax)
        v_ref[...] = v
        s_ref[...] = jnp.broadcast_to(jnp.sum(v), (PR, EC))

    return pl.pallas_call(
        body,
        grid=(1,),
        in_specs=[pl.BlockSpec((ER, EC), lambda i: (0, 0))],
        out_specs=[pl.BlockSpec((ER, EC), lambda i: (0, 0)),
                   pl.BlockSpec((PR, EC), lambda i: (0, 0))],
        out_shape=[jax.ShapeDtypeStruct((ER, EC), jnp.float32),
                   jax.ShapeDtypeStruct((PR, EC), jnp.float32)],
    )(lk2)


def _sc_spmm(wh, src3, dst, vals, zeros):
    n, d = wh.shape
    nch = src3.shape[1]
    epw = nch * CK
    rpt = (n // NS) // 8 * 8
    tail = n - NS * rpt

    @functools.partial(
        pl.kernel,
        mesh=_mesh(),
        compiler_params=pltpu.CompilerParams(needs_layout_passes=False),
        out_type=jax.ShapeDtypeStruct((NC, n, d), jnp.float32),
        scratch_types=[
            pltpu.VMEM((CK,), jnp.int32),
            pltpu.VMEM((CK,), jnp.int32),
            pltpu.VMEM((epw,), jnp.int32),
            pltpu.VMEM((epw,), jnp.float32),
            pltpu.VMEM((CK, d), jnp.float32),
            pltpu.VMEM((CK, d), jnp.float32),
            pltpu.VMEM_SHARED((n, d), jnp.float32),
            pltpu.SemaphoreType.DMA,
            pltpu.SemaphoreType.DMA,
            pltpu.SemaphoreType.DMA,
            pltpu.SemaphoreType.DMA,
        ],
    )
    def k(wh_hbm, src_hbm, dst_hbm, vals_hbm, z_hbm, out_hbm,
          srck0, srck1, dst_v, vals_v, rows0_v, rows1_v, acc, sg0, sg1, ss0, ss1):
        cid = lax.axis_index("c")
        sid = lax.axis_index("s")
        wid = sid * NC + cid
        row0 = pl.multiple_of(sid * rpt, 8)
        pltpu.sync_copy(z_hbm.at[pl.ds(row0, rpt)], acc.at[pl.ds(row0, rpt)])

        @pl.when(sid == 0)
        def _zero_tail():
            pltpu.sync_copy(z_hbm.at[pl.ds(NS * rpt, tail)],
                            acc.at[pl.ds(NS * rpt, tail)])

        pltpu.sync_copy(dst_hbm.at[pl.ds(wid * epw, epw)], dst_v)
        pltpu.sync_copy(vals_hbm.at[pl.ds(wid * epw, epw)], vals_v)

        plsc.subcore_barrier()

        def gather_start(c, buf, sidx, sem):
            pltpu.async_copy(src_hbm.at[wid, c], sidx, sem)
            pltpu.async_copy(wh_hbm.at[dst_v.at[pl.ds(c * CK, CK)]], buf, sem)

        def gather_wait(c, buf, sidx, sem):
            pltpu.make_async_copy(src_hbm.at[wid, c], sidx, sem).wait()
            pltpu.make_async_copy(wh_hbm.at[dst_v.at[pl.ds(c * CK, CK)]],
                                  buf, sem).wait()

        def scat_start(buf, sidx, sem):
            pltpu.async_copy(buf, acc.at[sidx], sem, add=True)

        def scat_wait(buf, sidx, sem):
            pltpu.make_async_copy(buf, acc.at[sidx], sem).wait()

        def scale(c, buf):
            def gstep(g, _):
                vv = vals_v[pl.ds(c * CK + g * L, L)]
                for i in range(L):
                    v = vv[i]
                    r = g * L + i
                    for q in range(d // L):
                        buf[r, pl.ds(q * L, L)] = buf[r, pl.ds(q * L, L)] * v
                return 0

            lax.fori_loop(0, CK // L, gstep, 0)

        gather_start(0, rows0_v, srck0, sg0)
        gather_start(1, rows1_v, srck1, sg1)

        def pair(p, _):
            c0 = p * 2
            c1 = c0 + 1
            gather_wait(c0, rows0_v, srck0, sg0)
            scale(c0, rows0_v)
            scat_start(rows0_v, srck0, ss0)

            @pl.when(c1 < nch)
            def _do1():
                gather_wait(c1, rows1_v, srck1, sg1)
                scale(c1, rows1_v)
                scat_start(rows1_v, srck1, ss1)

            @pl.when(c0 + 2 < nch)
            def _next0():
                scat_wait(rows0_v, srck0, ss0)
                gather_start(c0 + 2, rows0_v, srck0, sg0)

            @pl.when(c1 + 2 < nch)
            def _next1():
                scat_wait(rows1_v, srck1, ss1)
                gather_start(c1 + 2, rows1_v, srck1, sg1)

            return 0

        lax.fori_loop(0, (nch + 1) // 2, pair, 0)
        scat_wait(rows0_v, srck0, ss0)
        scat_wait(rows1_v, srck1, ss1)
        plsc.subcore_barrier()
        pltpu.sync_copy(acc.at[pl.ds(row0, rpt)],
                        out_hbm.at[cid, pl.ds(row0, rpt)])

        @pl.when(sid == 0)
        def _dump_tail():
            pltpu.sync_copy(acc.at[pl.ds(NS * rpt, tail)],
                            out_hbm.at[cid, pl.ds(NS * rpt, tail)])

    return k(wh, src3, dst, vals, zeros)


def _tc_post(p0, p1, srep):
    n, d = p0.shape
    rb = 1000

    def body(a_ref, b_ref, s_ref, o_ref):
        s_tot = jnp.max(s_ref[...])
        x = (a_ref[...] + b_ref[...]) / s_tot
        o_ref[...] = jnp.where(x > 0, x, jnp.exp(x) - 1.0)

    return pl.pallas_call(
        body,
        grid=(n // rb,),
        in_specs=[pl.BlockSpec((rb, d), lambda i: (i, 0)),
                  pl.BlockSpec((rb, d), lambda i: (i, 0)),
                  pl.BlockSpec(srep.shape, lambda i: (0, 0))],
        out_specs=pl.BlockSpec((rb, d), lambda i: (i, 0)),
        out_shape=jax.ShapeDtypeStruct((n, d), jnp.float32),
    )(p0, p1, srep)


def kernel(h, edge_index, W, a, nw):
    del nw
    n, _ = h.shape
    d = W.shape[1]
    e = edge_index.shape[1]
    a1 = a[0:d]
    a2 = a[d:2 * d]
    wh, s1, s2 = _tc_pre(h, W, a1, a2)
    s1 = s1.reshape(n)
    s2 = s2.reshape(n)
    src = edge_index[0]
    dst = edge_index[1]
    lk = _sc_edge(s1, s2, src, dst)
    lk2 = lk.reshape(ER, EC)
    vals2, srep = _tc_vals(lk2)
    vals = vals2.reshape(e)
    nch = e // NWK // CK
    src3 = src.reshape(NWK, nch, CK)
    zeros = jnp.zeros((n, d), jnp.float32)
    acc2 = _sc_spmm(wh, src3, dst, vals, zeros)
    return _tc_post(acc2[0], acc2[1], srep)


# speedup vs baseline: 28.1861x; 1.1906x over previous
import functools

import jax
import jax.numpy as jnp
from jax import lax
from jax.experimental import pallas as pl
from jax.experimental.pallas import tpu as pltpu
from jax.experimental.pallas import tpu_sc as plsc

NC = 2
NS = 16
L = 16
NWK = NC * NS
ALPHA = 0.2
CK = 80
ER = 2500
EC = 128
EBLK = 2500
PR = 8


def _mesh():
    return plsc.VectorSubcoreMesh(
        core_axis_name="c", subcore_axis_name="s", num_cores=NC, num_subcores=NS
    )


def _tc_pre(h, W, a1, a2):
    n, d_in = h.shape
    d = W.shape[1]
    rb = 1000

    def body(h_ref, w_ref, a1_ref, a2_ref, wh_ref, s1_ref, s2_ref):
        wh = jnp.dot(h_ref[...], w_ref[...], preferred_element_type=jnp.float32)
        wh_ref[...] = wh
        s1_ref[...] = jnp.dot(wh, a1_ref[...], preferred_element_type=jnp.float32)
        s2_ref[...] = jnp.dot(wh, a2_ref[...], preferred_element_type=jnp.float32)

    return pl.pallas_call(
        body,
        grid=(n // rb,),
        in_specs=[
            pl.BlockSpec((rb, d_in), lambda i: (i, 0)),
            pl.BlockSpec((d_in, d), lambda i: (0, 0)),
            pl.BlockSpec((d, 1), lambda i: (0, 0)),
            pl.BlockSpec((d, 1), lambda i: (0, 0)),
        ],
        out_specs=[
            pl.BlockSpec((rb, d), lambda i: (i, 0)),
            pl.BlockSpec((rb, 1), lambda i: (i, 0)),
            pl.BlockSpec((rb, 1), lambda i: (i, 0)),
        ],
        out_shape=[
            jax.ShapeDtypeStruct((n, d), jnp.float32),
            jax.ShapeDtypeStruct((n, 1), jnp.float32),
            jax.ShapeDtypeStruct((n, 1), jnp.float32),
        ],
    )(h, W, a1, a2)


def _sc_edge(s1, s2, src, dst):
    e = src.shape[0]
    epw = e // NWK
    n = s1.shape[0]

    @functools.partial(
        pl.kernel,
        mesh=_mesh(),
        compiler_params=pltpu.CompilerParams(needs_layout_passes=False),
        out_type=jax.ShapeDtypeStruct((e,), jnp.float32),
        scratch_types=[
            pltpu.VMEM((n,), jnp.float32),
            pltpu.VMEM((n,), jnp.float32),
            pltpu.VMEM((epw,), jnp.int32),
            pltpu.VMEM((epw,), jnp.int32),
            pltpu.VMEM((epw,), jnp.float32),
        ],
    )
    def k(s1_hbm, s2_hbm, src_hbm, dst_hbm, lk_hbm,
          s1_v, s2_v, src_v, dst_v, lk_v):
        wid = lax.axis_index("s") * NC + lax.axis_index("c")
        base = wid * epw
        pltpu.sync_copy(s1_hbm, s1_v)
        pltpu.sync_copy(s2_hbm, s2_v)
        pltpu.sync_copy(src_hbm.at[pl.ds(base, epw)], src_v)
        pltpu.sync_copy(dst_hbm.at[pl.ds(base, epw)], dst_v)

        def step(i, carry):
            x1 = plsc.load_gather(s1_v, [src_v[pl.ds(i * L, L)]])
            x2 = plsc.load_gather(s2_v, [dst_v[pl.ds(i * L, L)]])
            ev = x1 + x2
            lk_v[pl.ds(i * L, L)] = jnp.where(ev > 0, -ev, -ALPHA * ev)
            return carry

        lax.fori_loop(0, epw // L, step, 0)
        pltpu.sync_copy(lk_v, lk_hbm.at[pl.ds(base, epw)])

    return k(s1, s2, src, dst)


def _tc_vals(lk2):
    def body(lk_ref, v_ref, s_ref):
        x = jnp.exp(lk_ref[...])
        gmax = jnp.max(x)
        v = jnp.exp(x - gmax)
        v_ref[...] = v
        s_ref[...] = jnp.broadcast_to(jnp.sum(v), (PR, EC))

    return pl.pallas_call(
        body,
        grid=(1,),
        in_specs=[pl.BlockSpec((ER, EC), lambda i: (0, 0))],
        out_specs=[pl.BlockSpec((ER, EC), lambda i: (0, 0)),
                   pl.BlockSpec((PR, EC), lambda i: (0, 0))],
        out_shape=[jax.ShapeDtypeStruct((ER, EC), jnp.float32),
                   jax.ShapeDtypeStruct((PR, EC), jnp.float32)],
    )(lk2)


def _sc_spmm(wh, src3, dst, vals):
    n, d = wh.shape
    nch = src3.shape[1]
    epw = nch * CK
    rpt = (n // NS) // 8 * 8
    tail = n - NS * rpt
    NB = 3

    @functools.partial(
        pl.kernel,
        mesh=_mesh(),
        compiler_params=pltpu.CompilerParams(needs_layout_passes=False),
        out_type=jax.ShapeDtypeStruct((NC, n, d), jnp.float32),
        scratch_types=(
            [pltpu.VMEM((epw,), jnp.int32)]
            + [pltpu.VMEM((CK, d), jnp.float32)] * NB
            + [pltpu.VMEM((CK,), jnp.int32)] * NB
            + [pltpu.VMEM((CK,), jnp.float32)] * NB
            + [pltpu.VMEM_SHARED((n, d), jnp.float32)]
            + [pltpu.SemaphoreType.DMA] * (2 * NB)
        ),
    )
    def k(wh_hbm, src_hbm, dst_hbm, vals_hbm, out_hbm,
          dst_v, r0, r1, r2, si0, si1, si2, vc0, vc1, vc2, acc,
          sg0, sg1, sg2, ss0, ss1, ss2):
        bufs = [(r0, si0, vc0, sg0, ss0), (r1, si1, vc1, sg1, ss1),
                (r2, si2, vc2, sg2, ss2)]
        cid = lax.axis_index("c")
        sid = lax.axis_index("s")
        wid = sid * NC + cid
        row0 = pl.multiple_of(sid * rpt, 8)

        def zrow(r, _):
            for q in range(d // L):
                r0[r, pl.ds(q * L, L)] = jnp.zeros((L,), jnp.float32)
            return 0

        lax.fori_loop(0, CK, zrow, 0)
        nfull = rpt // CK
        rem = rpt - nfull * CK
        for j in range(nfull):
            pltpu.sync_copy(r0, acc.at[pl.ds(row0 + j * CK, CK)])
        if rem:
            pltpu.sync_copy(r0.at[pl.ds(0, rem)],
                            acc.at[pl.ds(row0 + nfull * CK, rem)])

        @pl.when(sid == 0)
        def _zero_tail():
            pltpu.sync_copy(r0.at[pl.ds(0, tail)],
                            acc.at[pl.ds(NS * rpt, tail)])

        pltpu.sync_copy(dst_hbm.at[pl.ds(wid * epw, epw)], dst_v)

        plsc.subcore_barrier()

        def gather_start(c, b):
            buf, si, vc, sg, _ = bufs[b]
            pltpu.async_copy(src_hbm.at[wid, c], si, sg)
            pltpu.async_copy(vals_hbm.at[pl.ds(wid * epw + c * CK, CK)], vc, sg)
            pltpu.async_copy(wh_hbm.at[dst_v.at[pl.ds(c * CK, CK)]], buf, sg)

        def gather_wait(c, b):
            buf, si, vc, sg, _ = bufs[b]
            pltpu.make_async_copy(src_hbm.at[wid, c], si, sg).wait()
            pltpu.make_async_copy(vals_hbm.at[pl.ds(wid * epw + c * CK, CK)],
                                  vc, sg).wait()
            pltpu.make_async_copy(wh_hbm.at[dst_v.at[pl.ds(c * CK, CK)]],
                                  buf, sg).wait()

        def scat_start(b):
            buf, si, _, _, ss = bufs[b]
            pltpu.async_copy(buf, acc.at[si], ss, add=True)

        def scat_wait(b):
            buf, si, _, _, ss = bufs[b]
            pltpu.make_async_copy(buf, acc.at[si], ss).wait()

        def scale(b):
            buf, _, vc, _, _ = bufs[b]

            def gstep(g, _):
                vv = vc[pl.ds(g * L, L)]
                for i in range(L):
                    v = vv[i]
                    r = g * L + i
                    for q in range(d // L):
                        buf[r, pl.ds(q * L, L)] = buf[r, pl.ds(q * L, L)] * v
                return 0

            lax.fori_loop(0, CK // L, gstep, 0)

        for b in range(NB):
            gather_start(b, b)

        def group(p, _):
            for b in range(NB):
                c = p * NB + b

                @pl.when(c < nch)
                def _do(c=c, b=b):
                    gather_wait(c, b)
                    scale(b)
                    scat_start(b)

                @pl.when(c + NB < nch)
                def _nxt(c=c, b=b):
                    scat_wait(b)
                    gather_start(c + NB, b)

            return 0

        lax.fori_loop(0, (nch + NB - 1) // NB, group, 0)
        for b in range(NB):
            scat_wait(b)
        plsc.subcore_barrier()
        pltpu.sync_copy(acc.at[pl.ds(row0, rpt)],
                        out_hbm.at[cid, pl.ds(row0, rpt)])

        @pl.when(sid == 0)
        def _dump_tail():
            pltpu.sync_copy(acc.at[pl.ds(NS * rpt, tail)],
                            out_hbm.at[cid, pl.ds(NS * rpt, tail)])

    return k(wh, src3, dst, vals)


def _tc_post(p0, p1, srep):
    n, d = p0.shape
    rb = 1000

    def body(a_ref, b_ref, s_ref, o_ref):
        s_tot = jnp.max(s_ref[...])
        x = (a_ref[...] + b_ref[...]) / s_tot
        o_ref[...] = jnp.where(x > 0, x, jnp.exp(x) - 1.0)

    return pl.pallas_call(
        body,
        grid=(n // rb,),
        in_specs=[pl.BlockSpec((rb, d), lambda i: (i, 0)),
                  pl.BlockSpec((rb, d), lambda i: (i, 0)),
                  pl.BlockSpec(srep.shape, lambda i: (0, 0))],
        out_specs=pl.BlockSpec((rb, d), lambda i: (i, 0)),
        out_shape=jax.ShapeDtypeStruct((n, d), jnp.float32),
    )(p0, p1, srep)


def kernel(h, edge_index, W, a, nw):
    del nw
    n, _ = h.shape
    d = W.shape[1]
    e = edge_index.shape[1]
    a1 = a[0:d]
    a2 = a[d:2 * d]
    wh, s1, s2 = _tc_pre(h, W, a1, a2)
    s1 = s1.reshape(n)
    s2 = s2.reshape(n)
    src = edge_index[0]
    dst = edge_index[1]
    lk = _sc_edge(s1, s2, src, dst)
    lk2 = lk.reshape(ER, EC)
    vals2, srep = _tc_vals(lk2)
    vals = vals2.reshape(e)
    nch = e // NWK // CK
    src3 = src.reshape(NWK, nch, CK)
    acc2 = _sc_spmm(wh, src3, dst, vals)
    return _tc_post(acc2[0], acc2[1], srep)
